# baseline (speedup 1.0000x reference)
import jax
import jax.numpy as jnp
from jax.experimental import pallas as pl
from jax.experimental.pallas import tpu as pltpu

B, S, D = 1, 2048, 1024
E, K, F = 8, 2, 512
FS = 2048
EPS = 1e-6
RM = 0.22
T = B * S
NSH = 4
FSC = FS // NSH
NSTEPS = E + NSH

_TDOT = (((1,), (1,)), ((), ()))


def _fused_kernel(x_ref, rmsw_ref, gw_ref, wgu_ref, wd_ref,
                  sgu_ref, sd_ref, o_ref,
                  acc_ref, hb_ref, comb_ref):
    j = pl.program_id(0)

    @pl.when(j == 0)
    def _init():
        x = x_ref[...]
        var = jnp.mean(x * x, axis=-1, keepdims=True)
        h = x * jax.lax.rsqrt(var + EPS) * rmsw_ref[...]
        logits = jax.lax.dot_general(h, gw_ref[...], _TDOT,
                                     preferred_element_type=jnp.float32)
        lcols = jax.lax.broadcasted_iota(jnp.int32, (T, E), 1)
        v1 = jnp.max(logits, axis=1, keepdims=True)
        i1 = jnp.argmax(logits, axis=1).reshape(T, 1)
        masked = jnp.where(lcols == i1, -jnp.inf, logits)
        v2 = jnp.max(masked, axis=1, keepdims=True)
        i2 = jnp.argmax(masked, axis=1).reshape(T, 1)
        p1 = jax.nn.sigmoid(v1 - v2)
        comb_ref[...] = (jnp.where(lcols == i1, p1, 0.0)
                         + jnp.where(lcols == i2, 1.0 - p1, 0.0))
        hb_ref[...] = h.astype(jnp.bfloat16)

    @pl.when(j < E)
    def _expert():
        hb = hb_ref[...]
        gu = jax.lax.dot_general(hb, wgu_ref[0], _TDOT,
                                 preferred_element_type=jnp.float32)
        g = gu[:, :F]
        u = gu[:, F:]
        cols = jax.lax.broadcasted_iota(jnp.int32, (T, E), 1)
        w = jnp.sum(jnp.where(cols == j, comb_ref[...], 0.0),
                    axis=1, keepdims=True)
        inter = (jax.nn.silu(g) * u * w).astype(jnp.bfloat16)
        val = jax.lax.dot_general(inter, wd_ref[0], _TDOT,
                                  preferred_element_type=jnp.float32)

        @pl.when(j == 0)
        def _store():
            acc_ref[...] = val

        @pl.when(j > 0)
        def _accum():
            acc_ref[...] += val

    @pl.when(j >= E)
    def _shared():
        hb = hb_ref[...]
        gu = jax.lax.dot_general(hb, sgu_ref[0], _TDOT,
                                 preferred_element_type=jnp.float32)
        g = gu[:, :FSC]
        u = gu[:, FSC:]
        inter = (jax.nn.silu(g) * u).astype(jnp.bfloat16)
        acc_ref[...] += jax.lax.dot_general(
            inter, sd_ref[...], _TDOT, preferred_element_type=jnp.float32)

    @pl.when(j == NSTEPS - 1)
    def _fin():
        o_ref[...] = x_ref[...] + RM * acc_ref[...]


def kernel(hidden_states, rms_w, gate_w, w_gate, w_up, w_down,
           sh_gate, sh_up, sh_down):
    x = hidden_states.reshape(T, D)
    wgu = jnp.concatenate([w_gate, w_up], axis=1).astype(jnp.bfloat16)
    wd = w_down.astype(jnp.bfloat16)
    sgu = jnp.concatenate(
        [sh_gate.reshape(NSH, FSC, D), sh_up.reshape(NSH, FSC, D)],
        axis=1).astype(jnp.bfloat16)
    sd = sh_down.astype(jnp.bfloat16)

    out = pl.pallas_call(
        _fused_kernel,
        grid=(NSTEPS,),
        in_specs=[
            pl.BlockSpec((T, D), lambda j: (0, 0)),
            pl.BlockSpec((1, D), lambda j: (0, 0)),
            pl.BlockSpec((E, D), lambda j: (0, 0)),
            pl.BlockSpec((1, 2 * F, D), lambda j: (jnp.minimum(j, E - 1), 0, 0)),
            pl.BlockSpec((1, D, F), lambda j: (jnp.minimum(j, E - 1), 0, 0)),
            pl.BlockSpec((1, 2 * FSC, D), lambda j: (jnp.clip(j - E, 0, NSH - 1), 0, 0)),
            pl.BlockSpec((D, FSC), lambda j: (0, jnp.clip(j - E, 0, NSH - 1))),
        ],
        out_specs=pl.BlockSpec((T, D), lambda j: (0, 0)),
        out_shape=jax.ShapeDtypeStruct((T, D), jnp.float32),
        scratch_shapes=[
            pltpu.VMEM((T, D), jnp.float32),
            pltpu.VMEM((T, D), jnp.bfloat16),
            pltpu.VMEM((T, E), jnp.float32),
        ],
        compiler_params=pltpu.CompilerParams(
            dimension_semantics=("arbitrary",),
        ),
    )(x, rms_w.reshape(1, D), gate_w, wgu, wd, sgu, sd)
    return out.reshape(B, S, D)


# speedup vs baseline: 1.0816x; 1.0816x over previous
import jax
import jax.numpy as jnp
from jax.experimental import pallas as pl
from jax.experimental.pallas import tpu as pltpu

B, S, D = 1, 2048, 1024
E, K, F = 8, 2, 512
FS = 2048
EPS = 1e-6
RM = 0.22
T = B * S
NSH = 4
FSC = FS // NSH
NSTEPS = E + NSH

_TDOT = (((1,), (1,)), ((), ()))


def _fused_kernel(x_ref, rmsw_ref, gw_ref, wg_ref, wu_ref, wd_ref,
                  sg_ref, su_ref, sd_ref, o_ref,
                  acc_ref, hb_ref, comb_ref):
    j = pl.program_id(0)

    @pl.when(j == 0)
    def _init():
        x = x_ref[...]
        var = jnp.mean(x * x, axis=-1, keepdims=True)
        h = x * jax.lax.rsqrt(var + EPS) * rmsw_ref[...]
        logits = jax.lax.dot_general(h, gw_ref[...], _TDOT,
                                     preferred_element_type=jnp.float32)
        lcols = jax.lax.broadcasted_iota(jnp.int32, (T, E), 1)
        v1 = jnp.max(logits, axis=1, keepdims=True)
        i1 = jnp.argmax(logits, axis=1).reshape(T, 1)
        masked = jnp.where(lcols == i1, -jnp.inf, logits)
        v2 = jnp.max(masked, axis=1, keepdims=True)
        i2 = jnp.argmax(masked, axis=1).reshape(T, 1)
        p1 = jax.nn.sigmoid(v1 - v2)
        comb_ref[...] = (jnp.where(lcols == i1, p1, 0.0)
                         + jnp.where(lcols == i2, 1.0 - p1, 0.0))
        hb_ref[...] = h.astype(jnp.bfloat16)

    @pl.when(j < E)
    def _expert():
        hb = hb_ref[...]
        g = jax.lax.dot_general(hb, wg_ref[0], _TDOT,
                                preferred_element_type=jnp.float32)
        u = jax.lax.dot_general(hb, wu_ref[0], _TDOT,
                                preferred_element_type=jnp.float32)
        cols = jax.lax.broadcasted_iota(jnp.int32, (T, E), 1)
        w = jnp.sum(jnp.where(cols == j, comb_ref[...], 0.0),
                    axis=1, keepdims=True)
        inter = (jax.nn.silu(g) * u * w).astype(jnp.bfloat16)
        val = jax.lax.dot_general(inter, wd_ref[0], _TDOT,
                                  preferred_element_type=jnp.float32)

        @pl.when(j == 0)
        def _store():
            acc_ref[...] = val

        @pl.when(j > 0)
        def _accum():
            acc_ref[...] += val

    @pl.when(j >= E)
    def _shared():
        hb = hb_ref[...]
        g = jax.lax.dot_general(hb, sg_ref[...], _TDOT,
                                preferred_element_type=jnp.float32)
        u = jax.lax.dot_general(hb, su_ref[...], _TDOT,
                                preferred_element_type=jnp.float32)
        inter = (jax.nn.silu(g) * u).astype(jnp.bfloat16)
        acc_ref[...] += jax.lax.dot_general(
            inter, sd_ref[...], _TDOT, preferred_element_type=jnp.float32)

    @pl.when(j == NSTEPS - 1)
    def _fin():
        o_ref[...] = x_ref[...] + RM * acc_ref[...]


def kernel(hidden_states, rms_w, gate_w, w_gate, w_up, w_down,
           sh_gate, sh_up, sh_down):
    x = hidden_states.reshape(T, D)
    wg = w_gate.astype(jnp.bfloat16)
    wu = w_up.astype(jnp.bfloat16)
    wd = w_down.astype(jnp.bfloat16)
    sg = sh_gate.astype(jnp.bfloat16)
    su = sh_up.astype(jnp.bfloat16)
    sd = sh_down.astype(jnp.bfloat16)

    out = pl.pallas_call(
        _fused_kernel,
        grid=(NSTEPS,),
        in_specs=[
            pl.BlockSpec((T, D), lambda j: (0, 0)),
            pl.BlockSpec((1, D), lambda j: (0, 0)),
            pl.BlockSpec((E, D), lambda j: (0, 0)),
            pl.BlockSpec((1, F, D), lambda j: (jnp.minimum(j, E - 1), 0, 0)),
            pl.BlockSpec((1, F, D), lambda j: (jnp.minimum(j, E - 1), 0, 0)),
            pl.BlockSpec((1, D, F), lambda j: (jnp.minimum(j, E - 1), 0, 0)),
            pl.BlockSpec((FSC, D), lambda j: (jnp.clip(j - E, 0, NSH - 1), 0)),
            pl.BlockSpec((FSC, D), lambda j: (jnp.clip(j - E, 0, NSH - 1), 0)),
            pl.BlockSpec((D, FSC), lambda j: (0, jnp.clip(j - E, 0, NSH - 1))),
        ],
        out_specs=pl.BlockSpec((T, D), lambda j: (0, 0)),
        out_shape=jax.ShapeDtypeStruct((T, D), jnp.float32),
        scratch_shapes=[
            pltpu.VMEM((T, D), jnp.float32),
            pltpu.VMEM((T, D), jnp.bfloat16),
            pltpu.VMEM((T, E), jnp.float32),
        ],
        compiler_params=pltpu.CompilerParams(
            dimension_semantics=("arbitrary",),
        ),
    )(x, rms_w.reshape(1, D), gate_w, wg, wu, wd, sg, su, sd)
    return out.reshape(B, S, D)


# speedup vs baseline: 1.1230x; 1.0384x over previous
import jax
import jax.numpy as jnp
from jax.experimental import pallas as pl
from jax.experimental.pallas import tpu as pltpu

B, S, D = 1, 2048, 1024
E, K, F = 8, 2, 512
FS = 2048
EPS = 1e-6
RM = 0.22
T = B * S
NSH = 4
FSC = FS // NSH
NSTEPS = E + NSH

_TDOT = (((1,), (1,)), ((), ()))


def _fused_kernel(x_ref, rmsw_ref, gw_ref, wg_ref, wu_ref, wd_ref,
                  sg_ref, su_ref, sd_ref, o_ref,
                  acc_ref, hb_ref, comb_ref):
    j = pl.program_id(0)

    @pl.when(j == 0)
    def _init():
        x = x_ref[...]
        var = jnp.mean(x * x, axis=-1, keepdims=True)
        h = x * jax.lax.rsqrt(var + EPS) * rmsw_ref[...]
        logits = jax.lax.dot_general(h, gw_ref[...], _TDOT,
                                     preferred_element_type=jnp.float32)
        lcols = jax.lax.broadcasted_iota(jnp.int32, (T, E), 1)
        v1 = jnp.max(logits, axis=1, keepdims=True)
        i1 = jnp.argmax(logits, axis=1).reshape(T, 1)
        masked = jnp.where(lcols == i1, -jnp.inf, logits)
        v2 = jnp.max(masked, axis=1, keepdims=True)
        i2 = jnp.argmax(masked, axis=1).reshape(T, 1)
        p1 = jax.nn.sigmoid(v1 - v2)
        comb_ref[...] = (jnp.where(lcols == i1, p1, 0.0)
                         + jnp.where(lcols == i2, 1.0 - p1, 0.0))
        hb_ref[...] = h.astype(jnp.bfloat16)
        acc_ref[...] = jnp.zeros_like(acc_ref)

    @pl.when(j < E)
    def _expert():
        hb = hb_ref[...]
        g = jax.lax.dot_general(hb, wg_ref[0], _TDOT,
                                preferred_element_type=jnp.float32)
        u = jax.lax.dot_general(hb, wu_ref[0], _TDOT,
                                preferred_element_type=jnp.float32)
        cols = jax.lax.broadcasted_iota(jnp.int32, (T, E), 1)
        w = jnp.sum(jnp.where(cols == j, comb_ref[...], 0.0),
                    axis=1, keepdims=True)
        inter = (jax.nn.silu(g) * u * w).astype(jnp.bfloat16)
        acc_ref[...] += jax.lax.dot_general(
            inter, wd_ref[0], _TDOT, preferred_element_type=jnp.float32)

    @pl.when(j >= E)
    def _shared():
        hb = hb_ref[...]
        g = jax.lax.dot_general(hb, sg_ref[...], _TDOT,
                                preferred_element_type=jnp.float32)
        u = jax.lax.dot_general(hb, su_ref[...], _TDOT,
                                preferred_element_type=jnp.float32)
        inter = (jax.nn.silu(g) * u).astype(jnp.bfloat16)
        acc_ref[...] += jax.lax.dot_general(
            inter, sd_ref[...], _TDOT, preferred_element_type=jnp.float32)

    @pl.when(j == NSTEPS - 1)
    def _fin():
        o_ref[...] = x_ref[...] + RM * acc_ref[...]


def kernel(hidden_states, rms_w, gate_w, w_gate, w_up, w_down,
           sh_gate, sh_up, sh_down):
    x = hidden_states.reshape(T, D)
    wg = w_gate.astype(jnp.bfloat16)
    wu = w_up.astype(jnp.bfloat16)
    wd = w_down.astype(jnp.bfloat16)
    sg = sh_gate.astype(jnp.bfloat16)
    su = sh_up.astype(jnp.bfloat16)
    sd = sh_down.astype(jnp.bfloat16)

    out = pl.pallas_call(
        _fused_kernel,
        grid=(NSTEPS,),
        in_specs=[
            pl.BlockSpec((T, D), lambda j: (0, 0)),
            pl.BlockSpec((1, D), lambda j: (0, 0)),
            pl.BlockSpec((E, D), lambda j: (0, 0)),
            pl.BlockSpec((1, F, D), lambda j: (jnp.minimum(j, E - 1), 0, 0)),
            pl.BlockSpec((1, F, D), lambda j: (jnp.minimum(j, E - 1), 0, 0)),
            pl.BlockSpec((1, D, F), lambda j: (jnp.minimum(j, E - 1), 0, 0)),
            pl.BlockSpec((FSC, D), lambda j: (jnp.clip(j - E, 0, NSH - 1), 0)),
            pl.BlockSpec((FSC, D), lambda j: (jnp.clip(j - E, 0, NSH - 1), 0)),
            pl.BlockSpec((D, FSC), lambda j: (0, jnp.clip(j - E, 0, NSH - 1))),
        ],
        out_specs=pl.BlockSpec((T, D), lambda j: (0, 0)),
        out_shape=jax.ShapeDtypeStruct((T, D), jnp.float32),
        scratch_shapes=[
            pltpu.VMEM((T, D), jnp.float32),
            pltpu.VMEM((T, D), jnp.bfloat16),
            pltpu.VMEM((T, E), jnp.float32),
        ],
        compiler_params=pltpu.CompilerParams(
            dimension_semantics=("arbitrary",),
        ),
    )(x, rms_w.reshape(1, D), gate_w, wg, wu, wd, sg, su, sd)
    return out.reshape(B, S, D)
